# baseline (speedup 1.0000x reference)
import jax
import jax.numpy as jnp
from jax import lax
from jax.experimental import pallas as pl
from jax.experimental.pallas import tpu as pltpu

_N, _A, _M, _C = 8, 8500, 200, 80
_LEVEL_STARTS = (0, 6400, 8000, 8400)
_LEVEL_ENDS = (6400, 8000, 8400, 8500)
_NLVL = 4
_AVG_BETA = 0.1
_SCORE_THR = 0.0
_TOPK = 2
_BG = 80
_EPS = 1e-5
_BIG = 10000.0
_TA = 1024


def _prep_kernel(ps_ref, pbt_ref, gtb_ref, pad_ref, top2_ref, sums):
    n = pl.program_id(0)
    st = ps_ref[0]
    maxc = jnp.max(st, axis=1, keepdims=True)
    flag = (jnp.transpose(maxc) > _SCORE_THR).astype(
        jnp.float32)
    pbt = pbt_ref[0]
    fw = (pbt[2:3, :] - pbt[0:1, :]) * flag
    fh = (pbt[3:4, :] - pbt[1:2, :]) * flag
    a_iota = lax.broadcasted_iota(jnp.int32, (1, _A), 1)
    for l in range(_NLVL):
        m = ((a_iota >= _LEVEL_STARTS[l]) &
             (a_iota < _LEVEL_ENDS[l])).astype(jnp.float32)
        cnt = jnp.sum(flag * m)
        sw = jnp.sum(fw * m)
        sh = jnp.sum(fh * m)
        prev_c = jnp.where(n == 0, 0.0, sums[0, l])
        prev_w = jnp.where(n == 0, 0.0, sums[1, l])
        prev_h = jnp.where(n == 0, 0.0, sums[2, l])
        sums[0, l] = prev_c + cnt
        sums[1, l] = prev_w + sw
        sums[2, l] = prev_h + sh

    @pl.when(n == _N - 1)
    def _finalize():
        avg_w, avg_h = [], []
        for l in range(_NLVL):
            cnt = sums[0, l]
            mw = sums[1, l] / jnp.maximum(cnt, 1.0)
            mh = sums[2, l] / jnp.maximum(cnt, 1.0)
            avg_w.append(jnp.where(cnt > 0, mw + _AVG_BETA * (0.0 - mw), 0.0))
            avg_h.append(jnp.where(cnt > 0, mh + _AVG_BETA * (0.0 - mh), 0.0))
        for nn in range(_N):
            gtb = gtb_ref[nn]
            gw = gtb[:, 2:3] - gtb[:, 0:1]
            gh = gtb[:, 3:4] - gtb[:, 1:2]
            pad = pad_ref[nn]
            cost = [avg_w[l] / (gw + _EPS) + gw / (avg_w[l] + _EPS) +
                    avg_h[l] / (gh + _EPS) + gh / (avg_h[l] + _EPS)
                    for l in range(_NLVL)]
            cols = []
            for l in range(_NLVL):
                rank = jnp.zeros((_M, 1), jnp.float32)
                for q in range(_NLVL):
                    lt = (cost[q] < cost[l]).astype(jnp.float32)
                    if q < l:
                        lt = lt + (cost[q] == cost[l]).astype(jnp.float32)
                    rank = rank + lt
                cols.append((rank < _TOPK).astype(jnp.float32) * pad)
            top2_ref[nn] = jnp.concatenate(cols, axis=1)


def _assign_kernel(ct_ref, gtl_ref, gtb_ref, top2_ref, pbt_ref,
                   lab_ref, box_ref, sco_ref, mask_ref):
    j = pl.program_id(1)
    gtb = gtb_ref[0]
    top2 = top2_ref[0]
    t0, t1, t2, t3 = (top2[:, 0:1], top2[:, 1:2],
                      top2[:, 2:3], top2[:, 3:4])

    a_glob = lax.broadcasted_iota(jnp.int32, (1, _TA), 1) + j * _TA
    lvl0 = a_glob < _LEVEL_ENDS[0]
    lvl1 = a_glob < _LEVEL_ENDS[1]
    lvl2 = a_glob < _LEVEL_ENDS[2]
    sel = jnp.where(lvl0, t0, jnp.where(lvl1, t1, jnp.where(lvl2, t2, t3)))
    selb = sel > 0.5

    x = ct_ref[0:1, :]
    y = ct_ref[1:2, :]
    pb = pbt_ref[0]
    px0, py0, px1, py1 = pb[0:1, :], pb[1:2, :], pb[2:3, :], pb[3:4, :]
    gx0, gy0, gx1, gy1 = gtb[:, 0:1], gtb[:, 1:2], gtb[:, 2:3], gtb[:, 3:4]

    in_gt = (x > gx0) & (x < gx1) & (y > gy0) & (y < gy1)
    valid = in_gt & selb

    xmin = jnp.maximum(gx0, px0)
    ymin = jnp.maximum(gy0, py0)
    xmax = jnp.minimum(gx1, px1)
    ymax = jnp.minimum(gy1, py1)
    inter = jnp.maximum(xmax - xmin, 0.0) * jnp.maximum(ymax - ymin, 0.0)
    a1 = (gx1 - gx0) * (gy1 - gy0)
    a2 = (px1 - px0) * (py1 - py0)
    denom = a1 + a2 - inter + 1e-09
    cost = jnp.where(valid, 1.0 - inter / denom, _BIG)

    min_cost = jnp.min(cost, axis=0, keepdims=True)
    pack = lax.broadcasted_iota(jnp.int32, (_M, 1), 0) * 128 + gtl_ref[0]
    packed = jnp.min(jnp.where(cost == min_cost, pack, _M * 128),
                     axis=0, keepdims=True)
    idx = lax.shift_right_logical(packed, 7)
    labm = packed & 127
    neg = min_cost > 0.5

    labels = jnp.where(neg, _BG, labm)
    lab_ref[0] = labels

    idx_eff = jnp.where(neg, -1, idx)
    m_iota = lax.broadcasted_iota(jnp.int32, (_M, _TA), 0)
    maskf = (m_iota == idx_eff).astype(jnp.float32)
    mask_ref[0] = maskf

    box_ref[0] = lax.dot_general(maskf, gtb, (((0,), (0,)), ((), ())))

    w = jnp.where(neg, 0.0, 1.0 - min_cost)
    w_t = jnp.transpose(w)
    lab_t = jnp.transpose(labels)
    c_iota = lax.broadcasted_iota(jnp.int32, (1, _C), 1)
    sco_ref[0] = jnp.where(lab_t == c_iota, w_t, 0.0)


def kernel(centers, num_anchors_list, gt_labels, gt_bboxes, pad_gt_mask,
           bg_index, pred_bboxes, pred_scores):
    del num_anchors_list, bg_index

    pred_bboxes_t = jnp.transpose(pred_bboxes, (0, 2, 1))
    centers_t = centers.T

    top2 = pl.pallas_call(
        _prep_kernel,
        grid=(_N,),
        in_specs=[
            pl.BlockSpec((1, _A, _C), lambda n: (n, 0, 0)),
            pl.BlockSpec((1, 4, _A), lambda n: (n, 0, 0)),
            pl.BlockSpec((_N, _M, 4), lambda n: (0, 0, 0)),
            pl.BlockSpec((_N, _M, 1), lambda n: (0, 0, 0)),
        ],
        out_specs=pl.BlockSpec((_N, _M, _NLVL), lambda n: (0, 0, 0)),
        out_shape=jax.ShapeDtypeStruct((_N, _M, _NLVL), jnp.float32),
        scratch_shapes=[pltpu.SMEM((3, _NLVL), jnp.float32)],
        compiler_params=pltpu.CompilerParams(
            dimension_semantics=("arbitrary",)),
    )(pred_scores, pred_bboxes_t, gt_bboxes, pad_gt_mask)

    nj = pl.cdiv(_A, _TA)
    labels3, bboxes, scores, mask_pos = pl.pallas_call(
        _assign_kernel,
        grid=(_N, nj),
        in_specs=[
            pl.BlockSpec((2, _TA), lambda n, j: (0, j)),
            pl.BlockSpec((1, _M, 1), lambda n, j: (n, 0, 0)),
            pl.BlockSpec((1, _M, 4), lambda n, j: (n, 0, 0)),
            pl.BlockSpec((1, _M, _NLVL), lambda n, j: (n, 0, 0)),
            pl.BlockSpec((1, 4, _TA), lambda n, j: (n, 0, j)),
        ],
        out_specs=[
            pl.BlockSpec((1, 1, _TA), lambda n, j: (n, 0, j)),
            pl.BlockSpec((1, _TA, 4), lambda n, j: (n, j, 0)),
            pl.BlockSpec((1, _TA, _C), lambda n, j: (n, j, 0)),
            pl.BlockSpec((1, _M, _TA), lambda n, j: (n, 0, j)),
        ],
        out_shape=[
            jax.ShapeDtypeStruct((_N, 1, _A), jnp.int32),
            jax.ShapeDtypeStruct((_N, _A, 4), jnp.float32),
            jax.ShapeDtypeStruct((_N, _A, _C), jnp.float32),
            jax.ShapeDtypeStruct((_N, _M, _A), jnp.float32),
        ],
        compiler_params=pltpu.CompilerParams(
            dimension_semantics=("parallel", "parallel")),
    )(centers_t, gt_labels, gt_bboxes, top2, pred_bboxes_t)

    return labels3.reshape(_N, _A), bboxes, scores, mask_pos


# speedup vs baseline: 1.1610x; 1.1610x over previous
import jax
import jax.numpy as jnp
from jax import lax
from jax.experimental import pallas as pl
from jax.experimental.pallas import tpu as pltpu

_N, _A, _M, _C = 8, 8500, 200, 80
_LEVEL_STARTS = (0, 6400, 8000, 8400)
_LEVEL_ENDS = (6400, 8000, 8400, 8500)
_NLVL = 4
_AVG_BETA = 0.1
_SCORE_THR = 0.0
_TOPK = 2
_BG = 80
_EPS = 1e-5
_BIG = 10000.0
_TA = 1024


def _prep_kernel(pst_ref, pbt_ref, gtb_ref, pad_ref, top2_ref, sums):
    n = pl.program_id(0)
    st = pst_ref[0]
    flag = (jnp.max(st, axis=0, keepdims=True) > _SCORE_THR).astype(
        jnp.float32)
    pbt = pbt_ref[0]
    fw = (pbt[2:3, :] - pbt[0:1, :]) * flag
    fh = (pbt[3:4, :] - pbt[1:2, :]) * flag
    a_iota = lax.broadcasted_iota(jnp.int32, (1, _A), 1)
    for l in range(_NLVL):
        m = ((a_iota >= _LEVEL_STARTS[l]) &
             (a_iota < _LEVEL_ENDS[l])).astype(jnp.float32)
        cnt = jnp.sum(flag * m)
        sw = jnp.sum(fw * m)
        sh = jnp.sum(fh * m)
        prev_c = jnp.where(n == 0, 0.0, sums[0, l])
        prev_w = jnp.where(n == 0, 0.0, sums[1, l])
        prev_h = jnp.where(n == 0, 0.0, sums[2, l])
        sums[0, l] = prev_c + cnt
        sums[1, l] = prev_w + sw
        sums[2, l] = prev_h + sh

    @pl.when(n == _N - 1)
    def _finalize():
        avg_w, avg_h = [], []
        for l in range(_NLVL):
            cnt = sums[0, l]
            mw = sums[1, l] / jnp.maximum(cnt, 1.0)
            mh = sums[2, l] / jnp.maximum(cnt, 1.0)
            avg_w.append(jnp.where(cnt > 0, mw + _AVG_BETA * (0.0 - mw), 0.0))
            avg_h.append(jnp.where(cnt > 0, mh + _AVG_BETA * (0.0 - mh), 0.0))
        for nn in range(_N):
            gtb = gtb_ref[nn]
            gw = gtb[:, 2:3] - gtb[:, 0:1]
            gh = gtb[:, 3:4] - gtb[:, 1:2]
            pad = pad_ref[nn]
            cost = [avg_w[l] / (gw + _EPS) + gw / (avg_w[l] + _EPS) +
                    avg_h[l] / (gh + _EPS) + gh / (avg_h[l] + _EPS)
                    for l in range(_NLVL)]
            cols = []
            for l in range(_NLVL):
                rank = jnp.zeros((_M, 1), jnp.float32)
                for q in range(_NLVL):
                    lt = (cost[q] < cost[l]).astype(jnp.float32)
                    if q < l:
                        lt = lt + (cost[q] == cost[l]).astype(jnp.float32)
                    rank = rank + lt
                cols.append((rank < _TOPK).astype(jnp.float32) * pad)
            top2_ref[nn] = jnp.concatenate(cols, axis=1)


def _assign_kernel(ct_ref, gtl_ref, gtb_ref, top2_ref, pbt_ref,
                   lab_ref, box_ref, sco_ref, mask_ref):
    j = pl.program_id(1)
    gtb = gtb_ref[0]
    top2 = top2_ref[0]
    t0, t1, t2, t3 = (top2[:, 0:1], top2[:, 1:2],
                      top2[:, 2:3], top2[:, 3:4])

    a_glob = lax.broadcasted_iota(jnp.int32, (1, _TA), 1) + j * _TA
    lvl0 = a_glob < _LEVEL_ENDS[0]
    lvl1 = a_glob < _LEVEL_ENDS[1]
    lvl2 = a_glob < _LEVEL_ENDS[2]
    sel = jnp.where(lvl0, t0, jnp.where(lvl1, t1, jnp.where(lvl2, t2, t3)))
    selb = sel > 0.5

    x = ct_ref[0:1, :]
    y = ct_ref[1:2, :]
    pb = pbt_ref[0]
    px0, py0, px1, py1 = pb[0:1, :], pb[1:2, :], pb[2:3, :], pb[3:4, :]
    gx0, gy0, gx1, gy1 = gtb[:, 0:1], gtb[:, 1:2], gtb[:, 2:3], gtb[:, 3:4]

    in_gt = (x > gx0) & (x < gx1) & (y > gy0) & (y < gy1)
    valid = in_gt & selb

    xmin = jnp.maximum(gx0, px0)
    ymin = jnp.maximum(gy0, py0)
    xmax = jnp.minimum(gx1, px1)
    ymax = jnp.minimum(gy1, py1)
    inter = jnp.maximum(xmax - xmin, 0.0) * jnp.maximum(ymax - ymin, 0.0)
    a1 = (gx1 - gx0) * (gy1 - gy0)
    a2 = (px1 - px0) * (py1 - py0)
    denom = a1 + a2 - inter + 1e-09
    cost = jnp.where(valid, 1.0 - inter / denom, _BIG)

    min_cost = jnp.min(cost, axis=0, keepdims=True)
    pack = lax.broadcasted_iota(jnp.int32, (_M, 1), 0) * 128 + gtl_ref[0]
    packed = jnp.min(jnp.where(cost == min_cost, pack, _M * 128),
                     axis=0, keepdims=True)
    idx = lax.shift_right_logical(packed, 7)
    labm = packed & 127
    neg = min_cost > 0.5

    labels = jnp.where(neg, _BG, labm)
    lab_ref[0] = labels

    idx_eff = jnp.where(neg, -1, idx)
    m_iota = lax.broadcasted_iota(jnp.int32, (_M, _TA), 0)
    maskf = (m_iota == idx_eff).astype(jnp.float32)
    mask_ref[0] = maskf

    box_ref[0] = lax.dot_general(maskf, gtb, (((0,), (0,)), ((), ())))

    w = jnp.where(neg, 0.0, 1.0 - min_cost)
    w_t = jnp.transpose(w)
    lab_t = jnp.transpose(labels)
    c_iota = lax.broadcasted_iota(jnp.int32, (1, _C), 1)
    sco_ref[0] = jnp.where(lab_t == c_iota, w_t, 0.0)


def kernel(centers, num_anchors_list, gt_labels, gt_bboxes, pad_gt_mask,
           bg_index, pred_bboxes, pred_scores):
    del num_anchors_list, bg_index

    pred_scores_t = jnp.transpose(pred_scores, (0, 2, 1))
    pred_bboxes_t = jnp.transpose(pred_bboxes, (0, 2, 1))
    centers_t = centers.T

    top2 = pl.pallas_call(
        _prep_kernel,
        grid=(_N,),
        in_specs=[
            pl.BlockSpec((1, _C, _A), lambda n: (n, 0, 0)),
            pl.BlockSpec((1, 4, _A), lambda n: (n, 0, 0)),
            pl.BlockSpec((_N, _M, 4), lambda n: (0, 0, 0)),
            pl.BlockSpec((_N, _M, 1), lambda n: (0, 0, 0)),
        ],
        out_specs=pl.BlockSpec((_N, _M, _NLVL), lambda n: (0, 0, 0)),
        out_shape=jax.ShapeDtypeStruct((_N, _M, _NLVL), jnp.float32),
        scratch_shapes=[pltpu.SMEM((3, _NLVL), jnp.float32)],
        compiler_params=pltpu.CompilerParams(
            dimension_semantics=("arbitrary",)),
    )(pred_scores_t, pred_bboxes_t, gt_bboxes, pad_gt_mask)

    nj = pl.cdiv(_A, _TA)
    labels3, bboxes, scores, mask_pos = pl.pallas_call(
        _assign_kernel,
        grid=(_N, nj),
        in_specs=[
            pl.BlockSpec((2, _TA), lambda n, j: (0, j)),
            pl.BlockSpec((1, _M, 1), lambda n, j: (n, 0, 0)),
            pl.BlockSpec((1, _M, 4), lambda n, j: (n, 0, 0)),
            pl.BlockSpec((1, _M, _NLVL), lambda n, j: (n, 0, 0)),
            pl.BlockSpec((1, 4, _TA), lambda n, j: (n, 0, j)),
        ],
        out_specs=[
            pl.BlockSpec((1, 1, _TA), lambda n, j: (n, 0, j)),
            pl.BlockSpec((1, _TA, 4), lambda n, j: (n, j, 0)),
            pl.BlockSpec((1, _TA, _C), lambda n, j: (n, j, 0)),
            pl.BlockSpec((1, _M, _TA), lambda n, j: (n, 0, j)),
        ],
        out_shape=[
            jax.ShapeDtypeStruct((_N, 1, _A), jnp.int32),
            jax.ShapeDtypeStruct((_N, _A, 4), jnp.float32),
            jax.ShapeDtypeStruct((_N, _A, _C), jnp.float32),
            jax.ShapeDtypeStruct((_N, _M, _A), jnp.float32),
        ],
        compiler_params=pltpu.CompilerParams(
            dimension_semantics=("parallel", "parallel")),
    )(centers_t, gt_labels, gt_bboxes, top2, pred_bboxes_t)

    return labels3.reshape(_N, _A), bboxes, scores, mask_pos


# speedup vs baseline: 1.1912x; 1.0261x over previous
import jax
import jax.numpy as jnp
from jax import lax
from jax.experimental import pallas as pl
from jax.experimental.pallas import tpu as pltpu

_N, _A, _M, _C = 8, 8500, 200, 80
_LEVEL_STARTS = (0, 6400, 8000, 8400)
_LEVEL_ENDS = (6400, 8000, 8400, 8500)
_NLVL = 4
_AVG_BETA = 0.1
_SCORE_THR = 0.0
_TOPK = 2
_BG = 80
_EPS = 1e-5
_BIG = 10000.0
_TA = 1280


def _prep_kernel(pst_ref, pbt_ref, gtb_ref, pad_ref, top2_ref, sums):
    n = pl.program_id(0)
    st = pst_ref[0]
    flag = (jnp.max(st, axis=0, keepdims=True) > _SCORE_THR).astype(
        jnp.float32)
    pbt = pbt_ref[0]
    fw = (pbt[2:3, :] - pbt[0:1, :]) * flag
    fh = (pbt[3:4, :] - pbt[1:2, :]) * flag
    a_iota = lax.broadcasted_iota(jnp.int32, (1, _A), 1)
    for l in range(_NLVL):
        m = ((a_iota >= _LEVEL_STARTS[l]) &
             (a_iota < _LEVEL_ENDS[l])).astype(jnp.float32)
        cnt = jnp.sum(flag * m)
        sw = jnp.sum(fw * m)
        sh = jnp.sum(fh * m)
        prev_c = jnp.where(n == 0, 0.0, sums[0, l])
        prev_w = jnp.where(n == 0, 0.0, sums[1, l])
        prev_h = jnp.where(n == 0, 0.0, sums[2, l])
        sums[0, l] = prev_c + cnt
        sums[1, l] = prev_w + sw
        sums[2, l] = prev_h + sh

    @pl.when(n == _N - 1)
    def _finalize():
        avg_w, avg_h = [], []
        for l in range(_NLVL):
            cnt = sums[0, l]
            mw = sums[1, l] / jnp.maximum(cnt, 1.0)
            mh = sums[2, l] / jnp.maximum(cnt, 1.0)
            avg_w.append(jnp.where(cnt > 0, mw + _AVG_BETA * (0.0 - mw), 0.0))
            avg_h.append(jnp.where(cnt > 0, mh + _AVG_BETA * (0.0 - mh), 0.0))
        for nn in range(_N):
            gtb = gtb_ref[nn]
            gw = gtb[:, 2:3] - gtb[:, 0:1]
            gh = gtb[:, 3:4] - gtb[:, 1:2]
            pad = pad_ref[nn]
            cost = [avg_w[l] / (gw + _EPS) + gw / (avg_w[l] + _EPS) +
                    avg_h[l] / (gh + _EPS) + gh / (avg_h[l] + _EPS)
                    for l in range(_NLVL)]
            cols = []
            for l in range(_NLVL):
                rank = jnp.zeros((_M, 1), jnp.float32)
                for q in range(_NLVL):
                    lt = (cost[q] < cost[l]).astype(jnp.float32)
                    if q < l:
                        lt = lt + (cost[q] == cost[l]).astype(jnp.float32)
                    rank = rank + lt
                cols.append((rank < _TOPK).astype(jnp.float32) * pad)
            top2_ref[nn] = jnp.concatenate(cols, axis=1)


def _assign_kernel(ct_ref, gtl_ref, gtb_ref, top2_ref, pbt_ref,
                   lab_ref, box_ref, sco_ref, mask_ref):
    j = pl.program_id(1)
    gtb = gtb_ref[0]
    top2 = top2_ref[0]
    t0, t1, t2, t3 = (top2[:, 0:1], top2[:, 1:2],
                      top2[:, 2:3], top2[:, 3:4])

    a_glob = lax.broadcasted_iota(jnp.int32, (1, _TA), 1) + j * _TA
    lvl0 = a_glob < _LEVEL_ENDS[0]
    lvl1 = a_glob < _LEVEL_ENDS[1]
    lvl2 = a_glob < _LEVEL_ENDS[2]
    sel = jnp.where(lvl0, t0, jnp.where(lvl1, t1, jnp.where(lvl2, t2, t3)))
    selb = sel > 0.5

    x = ct_ref[0:1, :]
    y = ct_ref[1:2, :]
    pb = pbt_ref[0]
    px0, py0, px1, py1 = pb[0:1, :], pb[1:2, :], pb[2:3, :], pb[3:4, :]
    gx0, gy0, gx1, gy1 = gtb[:, 0:1], gtb[:, 1:2], gtb[:, 2:3], gtb[:, 3:4]

    in_gt = (x > gx0) & (x < gx1) & (y > gy0) & (y < gy1)
    valid = in_gt & selb

    xmin = jnp.maximum(gx0, px0)
    ymin = jnp.maximum(gy0, py0)
    xmax = jnp.minimum(gx1, px1)
    ymax = jnp.minimum(gy1, py1)
    inter = jnp.maximum(xmax - xmin, 0.0) * jnp.maximum(ymax - ymin, 0.0)
    a1 = (gx1 - gx0) * (gy1 - gy0)
    a2 = (px1 - px0) * (py1 - py0)
    denom = a1 + a2 - inter + 1e-09
    cost = jnp.where(valid, 1.0 - inter / denom, _BIG)

    min_cost = jnp.min(cost, axis=0, keepdims=True)
    pack = lax.broadcasted_iota(jnp.int32, (_M, 1), 0) * 128 + gtl_ref[0]
    packed = jnp.min(jnp.where(cost == min_cost, pack, _M * 128),
                     axis=0, keepdims=True)
    idx = lax.shift_right_logical(packed, 7)
    labm = packed & 127
    neg = min_cost > 0.5

    labels = jnp.where(neg, _BG, labm)
    lab_ref[0] = labels

    idx_eff = jnp.where(neg, -1, idx)
    m_iota = lax.broadcasted_iota(jnp.int32, (_M, _TA), 0)
    maskf = (m_iota == idx_eff).astype(jnp.float32)
    mask_ref[0] = maskf

    box_ref[0] = lax.dot_general(maskf, gtb, (((0,), (0,)), ((), ())))

    w = jnp.where(neg, 0.0, 1.0 - min_cost)
    w_t = jnp.transpose(w)
    lab_t = jnp.transpose(labels)
    c_iota = lax.broadcasted_iota(jnp.int32, (1, _C), 1)
    sco_ref[0] = jnp.where(lab_t == c_iota, w_t, 0.0)


def kernel(centers, num_anchors_list, gt_labels, gt_bboxes, pad_gt_mask,
           bg_index, pred_bboxes, pred_scores):
    del num_anchors_list, bg_index

    pred_scores_t = jnp.transpose(pred_scores, (0, 2, 1))
    pred_bboxes_t = jnp.transpose(pred_bboxes, (0, 2, 1))
    centers_t = centers.T

    top2 = pl.pallas_call(
        _prep_kernel,
        grid=(_N,),
        in_specs=[
            pl.BlockSpec((1, _C, _A), lambda n: (n, 0, 0)),
            pl.BlockSpec((1, 4, _A), lambda n: (n, 0, 0)),
            pl.BlockSpec((_N, _M, 4), lambda n: (0, 0, 0)),
            pl.BlockSpec((_N, _M, 1), lambda n: (0, 0, 0)),
        ],
        out_specs=pl.BlockSpec((_N, _M, _NLVL), lambda n: (0, 0, 0)),
        out_shape=jax.ShapeDtypeStruct((_N, _M, _NLVL), jnp.float32),
        scratch_shapes=[pltpu.SMEM((3, _NLVL), jnp.float32)],
        compiler_params=pltpu.CompilerParams(
            dimension_semantics=("arbitrary",)),
    )(pred_scores_t, pred_bboxes_t, gt_bboxes, pad_gt_mask)

    nj = pl.cdiv(_A, _TA)
    labels3, bboxes, scores, mask_pos = pl.pallas_call(
        _assign_kernel,
        grid=(_N, nj),
        in_specs=[
            pl.BlockSpec((2, _TA), lambda n, j: (0, j)),
            pl.BlockSpec((1, _M, 1), lambda n, j: (n, 0, 0)),
            pl.BlockSpec((1, _M, 4), lambda n, j: (n, 0, 0)),
            pl.BlockSpec((1, _M, _NLVL), lambda n, j: (n, 0, 0)),
            pl.BlockSpec((1, 4, _TA), lambda n, j: (n, 0, j)),
        ],
        out_specs=[
            pl.BlockSpec((1, 1, _TA), lambda n, j: (n, 0, j)),
            pl.BlockSpec((1, _TA, 4), lambda n, j: (n, j, 0)),
            pl.BlockSpec((1, _TA, _C), lambda n, j: (n, j, 0)),
            pl.BlockSpec((1, _M, _TA), lambda n, j: (n, 0, j)),
        ],
        out_shape=[
            jax.ShapeDtypeStruct((_N, 1, _A), jnp.int32),
            jax.ShapeDtypeStruct((_N, _A, 4), jnp.float32),
            jax.ShapeDtypeStruct((_N, _A, _C), jnp.float32),
            jax.ShapeDtypeStruct((_N, _M, _A), jnp.float32),
        ],
        compiler_params=pltpu.CompilerParams(
            dimension_semantics=("parallel", "parallel")),
    )(centers_t, gt_labels, gt_bboxes, top2, pred_bboxes_t)

    return labels3.reshape(_N, _A), bboxes, scores, mask_pos
